# baseline (speedup 1.0000x reference)
import functools

import jax
import jax.numpy as jnp
from jax.experimental import pallas as pl
from jax.experimental.pallas import tpu as pltpu


def _fused_body(rgb_ref, hha_ref, gate_ref, out_ref, *, n_in, inv_hw):
    rgb = rgb_ref[...]
    hha = hha_ref[...]

    sr = jnp.sum(jnp.sum(rgb, axis=-1, dtype=jnp.float32), axis=-1)
    sh = jnp.sum(jnp.sum(hha, axis=-1, dtype=jnp.float32), axis=-1)
    s = jnp.concatenate([sr, sh], axis=-1)

    w1t = gate_ref[0:n_in, :]
    b1 = gate_ref[n_in:n_in + 1, :]
    w2d = gate_ref[n_in + 1:n_in + 2, :]
    b2d = gate_ref[n_in + 2:n_in + 3, 0:1]

    h = jnp.dot(s, w1t, preferred_element_type=jnp.float32)
    h = jnp.maximum(h * inv_hw + b1, 0.0)

    d = jnp.sum(h * w2d, axis=-1, keepdims=True) + b2d
    w_rgb = jax.nn.sigmoid(d)[:, :, None, None]

    out_ref[...] = (hha + w_rgb * (rgb - hha)).astype(out_ref.dtype)


def kernel(rgb, hha, w1, b1, w2, b2):
    assert rgb.shape == hha.shape and rgb.dtype == hha.dtype
    B, C, H, W = rgb.shape
    HID = w1.shape[0]
    n_in = 2 * C

    gate = jnp.concatenate([
        w1.T.astype(jnp.float32),
        b1.astype(jnp.float32).reshape(1, HID),
        (w2[0] - w2[1]).astype(jnp.float32).reshape(1, HID),
        jnp.full((1, HID), b2[0] - b2[1], dtype=jnp.float32),
    ], axis=0)

    NB = 4 if B % 4 == 0 else (2 if B % 2 == 0 else 1)

    body = functools.partial(_fused_body, n_in=n_in, inv_hw=1.0 / (H * W))
    return pl.pallas_call(
        body,
        out_shape=jax.ShapeDtypeStruct((B, C, H, W), rgb.dtype),
        grid=(B // NB,),
        in_specs=[
            pl.BlockSpec((NB, C, H, W), lambda b: (b, 0, 0, 0)),
            pl.BlockSpec((NB, C, H, W), lambda b: (b, 0, 0, 0)),
            pl.BlockSpec(gate.shape, lambda b: (0, 0)),
        ],
        out_specs=pl.BlockSpec((NB, C, H, W), lambda b: (b, 0, 0, 0)),
        compiler_params=pltpu.CompilerParams(
            dimension_semantics=("parallel",),
            vmem_limit_bytes=48 * 1024 * 1024),
        cost_estimate=pl.CostEstimate(
            flops=5 * B * C * H * W,
            transcendentals=B,
            bytes_accessed=3 * B * C * H * W * jnp.dtype(rgb.dtype).itemsize),
    )(rgb, hha, gate)
